# Initial kernel scaffold; baseline (speedup 1.0000x reference)
import jax
import jax.numpy as jnp
from jax.experimental import pallas as pl


def kernel(x, edge_index0, edge_index1, edge_index2, W1_0, b1_0, W1_1, b1_1, W1_2, b1_2, W2_0, b2_0, W2_1, b2_1, W2_2, b2_2):
    raise NotImplementedError("write your pallas kernel here")


# speedup vs baseline: 5.2082x; 5.2082x over previous
import functools

import jax
import jax.numpy as jnp
from jax import lax
from jax.experimental import pallas as pl
from jax.experimental.pallas import tpu as pltpu
from jax.experimental.pallas import tpu_sc as plsc

N = 10000
E = 160000
IN_F, HID_F, OUT_F = 256, 256, 128

_INFO = plsc.get_sparse_core_info()
NC = _INFO.num_cores
NS = _INFO.num_subcores
LANES = _INFO.num_lanes

K = 128
T = E // K
assert T * K == E

NP = 10240
CPS = NP // NS

RPS = (N // NS) & ~7
ZREM = N - RPS * NS

BN = 2000

_SC_PARAMS = pltpu.CompilerParams(needs_layout_passes=False)


def _norm(deg_col):
    return jnp.where(deg_col > 0, lax.rsqrt(jnp.maximum(deg_col, 1.0)), 0.0)


def _sc_bincount(idx_flat):
    mesh = plsc.VectorSubcoreMesh(core_axis_name="c", subcore_axis_name="s")

    @functools.partial(
        pl.kernel,
        out_type=jax.ShapeDtypeStruct((2, 3, 1, NP), jnp.float32),
        mesh=mesh,
        scratch_types=[
            pltpu.VMEM((K,), jnp.int32),
            pltpu.VMEM((NP,), jnp.float32),
            pltpu.VMEM((CPS,), jnp.float32),
            pltpu.VMEM((CPS,), jnp.float32),
            pltpu.VMEM_SHARED((NS, 1, NP), jnp.float32),
        ],
        compiler_params=_SC_PARAMS,
    )
    def kern(idx_hbm, deg_hbm, idx_v, acc, res_v, tmp_v, stage):
        c = lax.axis_index("c")
        s = lax.axis_index("s")
        q, ex = T // NS, T % NS
        start = s * q + jnp.minimum(s, ex)
        cnt = q + (s < ex).astype(jnp.int32)
        z16 = jnp.zeros((16,), jnp.float32)
        ones16 = jnp.ones((16,), jnp.float32)
        for j in range(3):
            def zero(i, _):
                acc[pl.ds(i * 16, 16)] = z16
                return 0

            lax.fori_loop(0, NP // 16, zero, 0)

            def chunk(i, _, j=j):
                base = (c * 3 + j) * E + (start + i) * K
                pltpu.sync_copy(idx_hbm.at[pl.ds(base, K)], idx_v)
                for g in range(8):
                    iv = idx_v[pl.ds(g * 16, 16)]
                    plsc.addupdate_scatter(acc, [iv], ones16)
                return 0

            lax.fori_loop(0, cnt, chunk, 0)
            pltpu.sync_copy(acc, stage.at[s, 0])
            plsc.subcore_barrier()
            pltpu.sync_copy(stage.at[0, 0, pl.ds(s * CPS, CPS)], res_v)
            for t in range(1, NS):
                pltpu.sync_copy(stage.at[t, 0, pl.ds(s * CPS, CPS)], tmp_v)

                def add(i, _):
                    res_v[pl.ds(i * 16, 16)] = (res_v[pl.ds(i * 16, 16)]
                                                + tmp_v[pl.ds(i * 16, 16)])
                    return 0

                lax.fori_loop(0, CPS // 16, add, 0)
            pltpu.sync_copy(res_v, deg_hbm.at[c, j, 0, pl.ds(s * CPS, CPS)])
            plsc.subcore_barrier()

    return kern(idx_flat)


def _sc_edge(tables, src_flat, dst_flat, zer_hbm, split_edges):
    fh = tables.shape[-1]
    mesh = plsc.VectorSubcoreMesh(core_axis_name="c", subcore_axis_name="s")
    wc = NC * NS if split_edges else NS

    @functools.partial(
        pl.kernel,
        out_type=jax.ShapeDtypeStruct((3, NC, N, fh), jnp.float32),
        mesh=mesh,
        scratch_types=[
            pltpu.VMEM((K,), jnp.int32),
            pltpu.VMEM((K,), jnp.int32),
            pltpu.VMEM((K, fh), jnp.float32),
            pltpu.VMEM_SHARED((N, fh), jnp.float32),
            pltpu.SemaphoreType.DMA,
        ],
        compiler_params=_SC_PARAMS,
    )
    def kern(tab_hbm, src_hbm, dst_hbm, zer_h, out_hbm,
             src_v, dst_v, rows_v, acc, sem):
        c = lax.axis_index("c")
        s = lax.axis_index("s")
        w = (s * NC + c) if split_edges else s
        q, ex = T // wc, T % wc
        start = w * q + jnp.minimum(w, ex)
        cnt = q + (w < ex).astype(jnp.int32)
        for r in range(3):
            tab_r = tab_hbm.at[r] if split_edges else tab_hbm.at[r, c]
            pltpu.sync_copy(zer_h.at[pl.ds(0, RPS)], acc.at[pl.ds(s * RPS, RPS)])

            @pl.when(s == 0)
            def _():
                pltpu.sync_copy(zer_h.at[pl.ds(0, ZREM)],
                                acc.at[pl.ds(NS * RPS, ZREM)])

            plsc.subcore_barrier()

            def chunk(i, _, r=r, tab_r=tab_r):
                base = r * E + (start + i) * K
                pltpu.sync_copy(src_hbm.at[pl.ds(base, K)], src_v)
                pltpu.sync_copy(dst_hbm.at[pl.ds(base, K)], dst_v)
                pltpu.async_copy(tab_r.at[src_v], rows_v, sem).wait()
                pltpu.sync_copy(rows_v, acc.at[dst_v], add=True)
                return 0

            lax.fori_loop(0, cnt, chunk, 0)
            plsc.subcore_barrier()
            pltpu.sync_copy(acc.at[pl.ds(s * RPS, RPS)],
                            out_hbm.at[r, c, pl.ds(s * RPS, RPS)])

            @pl.when(s == 0)
            def _(r=r):
                pltpu.sync_copy(acc.at[pl.ds(NS * RPS, ZREM)],
                                out_hbm.at[r, c, pl.ds(NS * RPS, ZREM)])

    return kern(tables, src_flat, dst_flat, zer_hbm)


def _tc_tables1(x, w1s, degt):
    def body(x_ref, w_ref, deg_ref, out_ref):
        xv = x_ref[...]
        for r in range(3):
            ns = _norm(deg_ref[:, r])
            t = jnp.dot(xv, w_ref[r], preferred_element_type=jnp.float32)
            t = t * ns[:, None]
            out_ref[r, 0] = t[:, : HID_F // 2]
            out_ref[r, 1] = t[:, HID_F // 2 :]

    return pl.pallas_call(
        body,
        grid=(N // BN,),
        in_specs=[
            pl.BlockSpec((BN, IN_F), lambda i: (i, 0)),
            pl.BlockSpec((3, IN_F, HID_F), lambda i: (0, 0, 0)),
            pl.BlockSpec((BN, 6), lambda i: (i, 0)),
        ],
        out_specs=pl.BlockSpec((3, 2, BN, HID_F // 2), lambda i: (0, 0, i, 0)),
        out_shape=jax.ShapeDtypeStruct((3, 2, N, HID_F // 2), jnp.float32),
    )(x, w1s, degt)


def _tc_mid(acc1, degt, w2s, b1s):
    def body(a_ref, deg_ref, w_ref, b_ref, out_ref):
        b1sum = b_ref[0] + b_ref[1] + b_ref[2]
        h0 = jnp.zeros((BN, HID_F // 2), jnp.float32)
        h1 = jnp.zeros((BN, HID_F // 2), jnp.float32)
        for r in range(3):
            nd = _norm(deg_ref[:, 3 + r])[:, None]
            h0 = h0 + a_ref[r, 0] * nd
            h1 = h1 + a_ref[r, 1] * nd
        h0 = jnp.maximum(h0 + b1sum[None, : HID_F // 2], 0.0)
        h1 = jnp.maximum(h1 + b1sum[None, HID_F // 2 :], 0.0)
        for r in range(3):
            ns = _norm(deg_ref[:, r])[:, None]
            t = (jnp.dot(h0, w_ref[r, : HID_F // 2], preferred_element_type=jnp.float32)
                 + jnp.dot(h1, w_ref[r, HID_F // 2 :], preferred_element_type=jnp.float32))
            out_ref[r] = t * ns

    return pl.pallas_call(
        body,
        grid=(N // BN,),
        in_specs=[
            pl.BlockSpec((3, 2, BN, HID_F // 2), lambda i: (0, 0, i, 0)),
            pl.BlockSpec((BN, 6), lambda i: (i, 0)),
            pl.BlockSpec((3, HID_F, OUT_F), lambda i: (0, 0, 0)),
            pl.BlockSpec((3, HID_F), lambda i: (0, 0)),
        ],
        out_specs=pl.BlockSpec((3, BN, OUT_F), lambda i: (0, i, 0)),
        out_shape=jax.ShapeDtypeStruct((3, N, OUT_F), jnp.float32),
    )(acc1, degt, w2s, b1s)


def _tc_final(acc2, degt, b2s):
    def body(a_ref, deg_ref, b_ref, out_ref):
        o = (b_ref[0] + b_ref[1] + b_ref[2])[None, :] * jnp.ones((BN, 1), jnp.float32)
        for r in range(3):
            nd = _norm(deg_ref[:, 3 + r])[:, None]
            o = o + (a_ref[r, 0] + a_ref[r, 1]) * nd
        out_ref[...] = o

    return pl.pallas_call(
        body,
        grid=(N // BN,),
        in_specs=[
            pl.BlockSpec((3, 2, BN, OUT_F), lambda i: (0, 0, i, 0)),
            pl.BlockSpec((BN, 6), lambda i: (i, 0)),
            pl.BlockSpec((3, OUT_F), lambda i: (0, 0)),
        ],
        out_specs=pl.BlockSpec((BN, OUT_F), lambda i: (i, 0)),
        out_shape=jax.ShapeDtypeStruct((N, OUT_F), jnp.float32),
    )(acc2, degt, b2s)


def kernel(x, edge_index0, edge_index1, edge_index2,
           W1_0, b1_0, W1_1, b1_1, W1_2, b1_2,
           W2_0, b2_0, W2_1, b2_1, W2_2, b2_2):
    src = jnp.concatenate([edge_index0[0], edge_index1[0], edge_index2[0]]).astype(jnp.int32)
    dst = jnp.concatenate([edge_index0[1], edge_index1[1], edge_index2[1]]).astype(jnp.int32)
    idx_flat = jnp.concatenate([src, dst])

    w1s = jnp.stack([W1_0, W1_1, W1_2])
    b1s = jnp.stack([b1_0, b1_1, b1_2])
    w2s = jnp.stack([W2_0, W2_1, W2_2])
    b2s = jnp.stack([b2_0, b2_1, b2_2])

    zer_h1 = jnp.zeros((RPS, HID_F // NC), jnp.float32)
    zer_h2 = jnp.zeros((RPS, OUT_F), jnp.float32)
    assert RPS >= ZREM

    deg = _sc_bincount(idx_flat)[:, :, 0, :N]
    degt = deg.reshape(6, N).T
    t1 = _tc_tables1(x, w1s, degt)
    acc1 = _sc_edge(t1, src, dst, zer_h1, split_edges=False)
    t2 = _tc_mid(acc1, degt, w2s, b1s)
    acc2 = _sc_edge(t2, src, dst, zer_h2, split_edges=True)
    return _tc_final(acc2, degt, b2s)


# speedup vs baseline: 8.2214x; 1.5786x over previous
import functools

import jax
import jax.numpy as jnp
from jax import lax
from jax.experimental import pallas as pl
from jax.experimental.pallas import tpu as pltpu
from jax.experimental.pallas import tpu_sc as plsc

N = 10000
E = 160000
IN_F, HID_F, OUT_F = 256, 256, 128

_INFO = plsc.get_sparse_core_info()
NC = _INFO.num_cores
NS = _INFO.num_subcores
LANES = _INFO.num_lanes

K = 128
T = E // K
assert T * K == E

NP = 10240
CPS = NP // NS

RPS = (N // NS) & ~7
ZREM = N - RPS * NS

BN = 2000

_SC_PARAMS = pltpu.CompilerParams(needs_layout_passes=False)


def _norm(deg_col):
    return jnp.where(deg_col > 0, lax.rsqrt(jnp.maximum(deg_col, 1.0)), 0.0)


_BK = 1280
_BT = E // _BK


def _sc_bincount(idx_flat):
    mesh = plsc.VectorSubcoreMesh(core_axis_name="c", subcore_axis_name="s")

    @functools.partial(
        pl.kernel,
        out_type=jax.ShapeDtypeStruct((2, 3, 1, NP), jnp.float32),
        mesh=mesh,
        scratch_types=[
            pltpu.VMEM((2, _BK), jnp.int32),
            pltpu.VMEM((NP,), jnp.float32),
            pltpu.VMEM((CPS,), jnp.float32),
            pltpu.VMEM((CPS,), jnp.float32),
            pltpu.VMEM_SHARED((NS, 1, NP), jnp.float32),
            pltpu.SemaphoreType.DMA,
            pltpu.SemaphoreType.DMA,
        ],
        compiler_params=_SC_PARAMS,
    )
    def kern(idx_hbm, deg_hbm, idx_v, acc, res_v, tmp_v, stage, sem0, sem1):
        c = lax.axis_index("c")
        s = lax.axis_index("s")
        q, ex = _BT // NS, _BT % NS
        start = s * q + jnp.minimum(s, ex)
        cnt = q + (s < ex).astype(jnp.int32)
        z16 = jnp.zeros((16,), jnp.float32)
        ones16 = jnp.ones((16,), jnp.float32)
        sems = (sem0, sem1)
        for j in range(3):
            def zero(i, _):
                acc[pl.ds(i * 16, 16)] = z16
                return 0

            lax.fori_loop(0, NP // 16, zero, 0)

            def count(b, i, j=j):
                for g in range(_BK // 16):
                    iv = idx_v[b, pl.ds(g * 16, 16)]
                    plsc.addupdate_scatter(acc, [iv], ones16)

            def load(b, i, j=j):
                base = (c * 3 + j) * E + (start + i) * _BK
                return pltpu.async_copy(idx_hbm.at[pl.ds(base, _BK)],
                                        idx_v.at[b], sems[b])

            @pl.when(cnt > 0)
            def _(j=j):
                load(0, 0).wait()

            def pair(p, _, j=j):
                d1 = load(1, 2 * p + 1)
                count(0, 2 * p)
                d1.wait()
                d0 = load(0, jnp.minimum(2 * p + 2, cnt - 1))
                count(1, 2 * p + 1)
                d0.wait()
                return 0

            lax.fori_loop(0, cnt // 2, pair, 0)

            @pl.when(cnt % 2 == 1)
            def _(j=j):
                count(0, cnt - 1)
            pltpu.sync_copy(acc, stage.at[s, 0])
            plsc.subcore_barrier()
            pltpu.sync_copy(stage.at[0, 0, pl.ds(s * CPS, CPS)], res_v)
            for t in range(1, NS):
                pltpu.sync_copy(stage.at[t, 0, pl.ds(s * CPS, CPS)], tmp_v)

                def add(i, _):
                    res_v[pl.ds(i * 16, 16)] = (res_v[pl.ds(i * 16, 16)]
                                                + tmp_v[pl.ds(i * 16, 16)])
                    return 0

                lax.fori_loop(0, CPS // 16, add, 0)
            pltpu.sync_copy(res_v, deg_hbm.at[c, j, 0, pl.ds(s * CPS, CPS)])
            plsc.subcore_barrier()

    return kern(idx_flat)


_G = 3


def _sc_edge(tables, sd, zer_hbm, split_edges):
    fh = tables.shape[-1]
    mesh = plsc.VectorSubcoreMesh(core_axis_name="c", subcore_axis_name="s")
    wc = NC * NS if split_edges else NS

    @functools.partial(
        pl.kernel,
        out_type=jax.ShapeDtypeStruct((3, NC, N, fh), jnp.float32),
        mesh=mesh,
        scratch_types=[
            pltpu.VMEM((_G, 2, K), jnp.int32),
            pltpu.VMEM((_G, K, fh), jnp.float32),
            pltpu.VMEM_SHARED((N, fh), jnp.float32),
            pltpu.SemaphoreType.DMA,
        ] + [pltpu.SemaphoreType.DMA] * (2 * _G),
        compiler_params=_SC_PARAMS,
    )
    def kern(tab_hbm, sd_hbm, zer_h, out_hbm, idx_v, rows_v, acc, isem, *gssems):
        gsems, ssems = gssems[:_G], gssems[_G:]
        c = lax.axis_index("c")
        s = lax.axis_index("s")
        w = (s * NC + c) if split_edges else s
        q, ex = T // wc, T % wc
        start = w * q + jnp.minimum(w, ex)
        cnt = q + (w < ex).astype(jnp.int32)
        for r in range(3):
            tab_r = tab_hbm.at[r] if split_edges else tab_hbm.at[r, c]
            pltpu.sync_copy(zer_h.at[pl.ds(0, RPS)], acc.at[pl.ds(s * RPS, RPS)])

            @pl.when(s == 0)
            def _():
                pltpu.sync_copy(zer_h.at[pl.ds(0, ZREM)],
                                acc.at[pl.ds(NS * RPS, ZREM)])

            plsc.subcore_barrier()

            def group(g, _, r=r, tab_r=tab_r):
                cid0 = r * T + start + g * _G
                pltpu.async_copy(sd_hbm.at[pl.ds(cid0, _G)], idx_v, isem).wait()
                dg = [pltpu.async_copy(tab_r.at[idx_v.at[k, 0]],
                                       rows_v.at[k], gsems[k])
                      for k in range(_G)]
                ds = []
                for k in range(_G):
                    dg[k].wait()
                    ds.append(pltpu.async_copy(rows_v.at[k],
                                               acc.at[idx_v.at[k, 1]],
                                               ssems[k], add=True))
                for d in ds:
                    d.wait()
                return 0

            lax.fori_loop(0, cnt // _G, group, 0)

            def tailchunk(i, _, r=r, tab_r=tab_r):
                cid = r * T + start + (cnt // _G) * _G + i
                pltpu.sync_copy(sd_hbm.at[cid], idx_v.at[0])
                pltpu.async_copy(tab_r.at[idx_v.at[0, 0]], rows_v.at[0],
                                 gsems[0]).wait()
                pltpu.async_copy(rows_v.at[0], acc.at[idx_v.at[0, 1]],
                                 ssems[0], add=True).wait()
                return 0

            lax.fori_loop(0, cnt % _G, tailchunk, 0)
            plsc.subcore_barrier()
            pltpu.sync_copy(acc.at[pl.ds(s * RPS, RPS)],
                            out_hbm.at[r, c, pl.ds(s * RPS, RPS)])

            @pl.when(s == 0)
            def _(r=r):
                pltpu.sync_copy(acc.at[pl.ds(NS * RPS, ZREM)],
                                out_hbm.at[r, c, pl.ds(NS * RPS, ZREM)])

    return kern(tables, sd, zer_hbm)


def _tc_tables1(x, w1s, degt):
    def body(x_ref, w_ref, deg_ref, out_ref):
        xv = x_ref[...]
        for r in range(3):
            ns = _norm(deg_ref[:, r])
            t = jnp.dot(xv, w_ref[r], preferred_element_type=jnp.float32)
            t = t * ns[:, None]
            out_ref[r, 0] = t[:, : HID_F // 2]
            out_ref[r, 1] = t[:, HID_F // 2 :]

    return pl.pallas_call(
        body,
        grid=(N // BN,),
        in_specs=[
            pl.BlockSpec((BN, IN_F), lambda i: (i, 0)),
            pl.BlockSpec((3, IN_F, HID_F), lambda i: (0, 0, 0)),
            pl.BlockSpec((BN, 6), lambda i: (i, 0)),
        ],
        out_specs=pl.BlockSpec((3, 2, BN, HID_F // 2), lambda i: (0, 0, i, 0)),
        out_shape=jax.ShapeDtypeStruct((3, 2, N, HID_F // 2), jnp.float32),
    )(x, w1s, degt)


def _tc_mid(acc1, degt, w2s, b1s):
    def body(a_ref, deg_ref, w_ref, b_ref, out_ref):
        b1sum = b_ref[0] + b_ref[1] + b_ref[2]
        h0 = jnp.zeros((BN, HID_F // 2), jnp.float32)
        h1 = jnp.zeros((BN, HID_F // 2), jnp.float32)
        for r in range(3):
            nd = _norm(deg_ref[:, 3 + r])[:, None]
            h0 = h0 + a_ref[r, 0] * nd
            h1 = h1 + a_ref[r, 1] * nd
        h0 = jnp.maximum(h0 + b1sum[None, : HID_F // 2], 0.0)
        h1 = jnp.maximum(h1 + b1sum[None, HID_F // 2 :], 0.0)
        for r in range(3):
            ns = _norm(deg_ref[:, r])[:, None]
            t = (jnp.dot(h0, w_ref[r, : HID_F // 2], preferred_element_type=jnp.float32)
                 + jnp.dot(h1, w_ref[r, HID_F // 2 :], preferred_element_type=jnp.float32))
            out_ref[r] = t * ns

    return pl.pallas_call(
        body,
        grid=(N // BN,),
        in_specs=[
            pl.BlockSpec((3, 2, BN, HID_F // 2), lambda i: (0, 0, i, 0)),
            pl.BlockSpec((BN, 6), lambda i: (i, 0)),
            pl.BlockSpec((3, HID_F, OUT_F), lambda i: (0, 0, 0)),
            pl.BlockSpec((3, HID_F), lambda i: (0, 0)),
        ],
        out_specs=pl.BlockSpec((3, BN, OUT_F), lambda i: (0, i, 0)),
        out_shape=jax.ShapeDtypeStruct((3, N, OUT_F), jnp.float32),
    )(acc1, degt, w2s, b1s)


def _tc_final(acc2, degt, b2s):
    def body(a_ref, deg_ref, b_ref, out_ref):
        o = (b_ref[0] + b_ref[1] + b_ref[2])[None, :] * jnp.ones((BN, 1), jnp.float32)
        for r in range(3):
            nd = _norm(deg_ref[:, 3 + r])[:, None]
            o = o + (a_ref[r, 0] + a_ref[r, 1]) * nd
        out_ref[...] = o

    return pl.pallas_call(
        body,
        grid=(N // BN,),
        in_specs=[
            pl.BlockSpec((3, 2, BN, OUT_F), lambda i: (0, 0, i, 0)),
            pl.BlockSpec((BN, 6), lambda i: (i, 0)),
            pl.BlockSpec((3, OUT_F), lambda i: (0, 0)),
        ],
        out_specs=pl.BlockSpec((BN, OUT_F), lambda i: (i, 0)),
        out_shape=jax.ShapeDtypeStruct((N, OUT_F), jnp.float32),
    )(acc2, degt, b2s)


def kernel(x, edge_index0, edge_index1, edge_index2,
           W1_0, b1_0, W1_1, b1_1, W1_2, b1_2,
           W2_0, b2_0, W2_1, b2_1, W2_2, b2_2):
    src = jnp.concatenate([edge_index0[0], edge_index1[0], edge_index2[0]]).astype(jnp.int32)
    dst = jnp.concatenate([edge_index0[1], edge_index1[1], edge_index2[1]]).astype(jnp.int32)
    idx_flat = jnp.concatenate([src, dst])
    sd = jnp.stack([src.reshape(3 * T, K), dst.reshape(3 * T, K)], axis=1)

    w1s = jnp.stack([W1_0, W1_1, W1_2])
    b1s = jnp.stack([b1_0, b1_1, b1_2])
    w2s = jnp.stack([W2_0, W2_1, W2_2])
    b2s = jnp.stack([b2_0, b2_1, b2_2])

    zer_h1 = jnp.zeros((RPS, HID_F // NC), jnp.float32)
    zer_h2 = jnp.zeros((RPS, OUT_F), jnp.float32)
    assert RPS >= ZREM

    deg = _sc_bincount(idx_flat)[:, :, 0, :N]
    degt = deg.reshape(6, N).T
    t1 = _tc_tables1(x, w1s, degt)
    acc1 = _sc_edge(t1, sd, zer_h1, split_edges=False)
    t2 = _tc_mid(acc1, degt, w2s, b1s)
    acc2 = _sc_edge(t2, sd, zer_h2, split_edges=True)
    return _tc_final(acc2, degt, b2s)


# speedup vs baseline: 10.2046x; 1.2412x over previous
import functools

import jax
import jax.numpy as jnp
from jax import lax
from jax.experimental import pallas as pl
from jax.experimental.pallas import tpu as pltpu
from jax.experimental.pallas import tpu_sc as plsc

N = 10000
E = 160000
IN_F, HID_F, OUT_F = 256, 256, 128

_INFO = plsc.get_sparse_core_info()
NC = _INFO.num_cores
NS = _INFO.num_subcores
LANES = _INFO.num_lanes

K = 128
T = E // K
assert T * K == E

NP = 10240
CPS = NP // NS

RPS = (N // NS) & ~7
ZREM = N - RPS * NS

BN = 2000

_SC_PARAMS = pltpu.CompilerParams(needs_layout_passes=False)


def _norm(deg_col):
    return jnp.where(deg_col > 0, lax.rsqrt(jnp.maximum(deg_col, 1.0)), 0.0)


_BK = 1280
_BT = E // _BK


def _sc_bincount(idx_flat):
    mesh = plsc.VectorSubcoreMesh(core_axis_name="c", subcore_axis_name="s")

    @functools.partial(
        pl.kernel,
        out_type=jax.ShapeDtypeStruct((2, 3, 1, NP), jnp.float32),
        mesh=mesh,
        scratch_types=[
            pltpu.VMEM((2, _BK), jnp.int32),
            pltpu.VMEM((NP,), jnp.float32),
            pltpu.VMEM((CPS,), jnp.float32),
            pltpu.VMEM((CPS,), jnp.float32),
            pltpu.VMEM_SHARED((NS, 1, NP), jnp.float32),
            pltpu.SemaphoreType.DMA,
            pltpu.SemaphoreType.DMA,
        ],
        compiler_params=_SC_PARAMS,
    )
    def kern(idx_hbm, deg_hbm, idx_v, acc, res_v, tmp_v, stage, sem0, sem1):
        c = lax.axis_index("c")
        s = lax.axis_index("s")
        q, ex = _BT // NS, _BT % NS
        start = s * q + jnp.minimum(s, ex)
        cnt = q + (s < ex).astype(jnp.int32)
        z16 = jnp.zeros((16,), jnp.float32)
        ones16 = jnp.ones((16,), jnp.float32)
        sems = (sem0, sem1)
        for j in range(3):
            def zero(i, _):
                acc[pl.ds(i * 16, 16)] = z16
                return 0

            lax.fori_loop(0, NP // 16, zero, 0)

            def count(b, i, j=j):
                for g in range(_BK // 16):
                    iv = idx_v[b, pl.ds(g * 16, 16)]
                    plsc.addupdate_scatter(acc, [iv], ones16)

            def load(b, i, j=j):
                base = (c * 3 + j) * E + (start + i) * _BK
                return pltpu.async_copy(idx_hbm.at[pl.ds(base, _BK)],
                                        idx_v.at[b], sems[b])

            @pl.when(cnt > 0)
            def _(j=j):
                load(0, 0).wait()

            def pair(p, _, j=j):
                d1 = load(1, 2 * p + 1)
                count(0, 2 * p)
                d1.wait()
                d0 = load(0, jnp.minimum(2 * p + 2, cnt - 1))
                count(1, 2 * p + 1)
                d0.wait()
                return 0

            lax.fori_loop(0, cnt // 2, pair, 0)

            @pl.when(cnt % 2 == 1)
            def _(j=j):
                count(0, cnt - 1)
            pltpu.sync_copy(acc, stage.at[s, 0])
            plsc.subcore_barrier()
            pltpu.sync_copy(stage.at[0, 0, pl.ds(s * CPS, CPS)], res_v)
            for t in range(1, NS):
                pltpu.sync_copy(stage.at[t, 0, pl.ds(s * CPS, CPS)], tmp_v)

                def add(i, _):
                    res_v[pl.ds(i * 16, 16)] = (res_v[pl.ds(i * 16, 16)]
                                                + tmp_v[pl.ds(i * 16, 16)])
                    return 0

                lax.fori_loop(0, CPS // 16, add, 0)
            pltpu.sync_copy(res_v, deg_hbm.at[c, j, 0, pl.ds(s * CPS, CPS)])
            plsc.subcore_barrier()

    return kern(idx_flat)


KE = 64
TE = E // KE
_G = 3


def _sc_edge(tables, sd, zer_hbm, split_edges):
    fh = tables.shape[-1]
    mesh = plsc.VectorSubcoreMesh(core_axis_name="c", subcore_axis_name="s")
    wc = NC * NS if split_edges else NS
    SL = 2 * _G

    @functools.partial(
        pl.kernel,
        out_type=jax.ShapeDtypeStruct((3, NC, N, fh), jnp.float32),
        mesh=mesh,
        scratch_types=[
            pltpu.VMEM((SL, 2, KE), jnp.int32),
            pltpu.VMEM((SL, KE, fh), jnp.float32),
            pltpu.VMEM_SHARED((N, fh), jnp.float32),
            pltpu.SemaphoreType.DMA,
        ] + [pltpu.SemaphoreType.DMA] * (2 * SL),
        compiler_params=_SC_PARAMS,
    )
    def kern(tab_hbm, sd_hbm, zer_h, out_hbm, idx_v, rows_v, acc, isem, *sems):
        gsems, ssems = sems[:SL], sems[SL:]
        c = lax.axis_index("c")
        s = lax.axis_index("s")
        w = (s * NC + c) if split_edges else s
        q, ex = TE // wc, TE % wc
        start = w * q + jnp.minimum(w, ex)
        cnt = q + (w < ex).astype(jnp.int32)
        ngroups = cnt // _G
        npairs = ngroups // 2
        for r in range(3):
            tab_r = tab_hbm.at[r] if split_edges else tab_hbm.at[r, c]
            pltpu.sync_copy(zer_h.at[pl.ds(0, RPS)], acc.at[pl.ds(s * RPS, RPS)])

            @pl.when(s == 0)
            def _():
                pltpu.sync_copy(zer_h.at[pl.ds(0, ZREM)],
                                acc.at[pl.ds(NS * RPS, ZREM)])

            plsc.subcore_barrier()

            def load_idx(bank, g, r=r):
                cid0 = r * TE + start + g * _G
                pltpu.async_copy(sd_hbm.at[pl.ds(cid0, _G)],
                                 idx_v.at[pl.ds(bank * _G, _G)], isem).wait()

            def issue_gathers(bank, tab_r=tab_r):
                for k in range(_G):
                    sl = bank * _G + k
                    pltpu.async_copy(tab_r.at[idx_v.at[sl, 0]],
                                     rows_v.at[sl], gsems[sl])

            def gathers_to_scatters(bank, tab_r=tab_r):
                for k in range(_G):
                    sl = bank * _G + k
                    pltpu.make_async_copy(tab_r.at[idx_v.at[sl, 0]],
                                          rows_v.at[sl], gsems[sl]).wait()
                    pltpu.async_copy(rows_v.at[sl], acc.at[idx_v.at[sl, 1]],
                                     ssems[sl], add=True)

            def drain_scatters(bank):
                for k in range(_G):
                    sl = bank * _G + k
                    pltpu.make_async_copy(rows_v.at[sl],
                                          acc.at[idx_v.at[sl, 1]],
                                          ssems[sl]).wait()

            @pl.when(ngroups > 0)
            def _():
                load_idx(0, 0)
                issue_gathers(0)

            def pair(p, _):
                @pl.when(p > 0)
                def _():
                    drain_scatters(1)

                load_idx(1, 2 * p + 1)
                gathers_to_scatters(0)
                issue_gathers(1)
                drain_scatters(0)

                @pl.when(2 * p + 2 < ngroups)
                def _():
                    load_idx(0, 2 * p + 2)
                    issue_gathers(0)

                gathers_to_scatters(1)
                return 0

            lax.fori_loop(0, npairs, pair, 0)

            @pl.when(ngroups % 2 == 1)
            def _():
                gathers_to_scatters(0)
                drain_scatters(0)

            @pl.when(npairs > 0)
            def _():
                drain_scatters(1)

            def tailchunk(i, _, r=r, tab_r=tab_r):
                cid = r * TE + start + ngroups * _G + i
                pltpu.sync_copy(sd_hbm.at[cid], idx_v.at[0])
                pltpu.async_copy(tab_r.at[idx_v.at[0, 0]], rows_v.at[0],
                                 gsems[0]).wait()
                pltpu.async_copy(rows_v.at[0], acc.at[idx_v.at[0, 1]],
                                 ssems[0], add=True).wait()
                return 0

            lax.fori_loop(0, cnt % _G, tailchunk, 0)
            plsc.subcore_barrier()
            pltpu.sync_copy(acc.at[pl.ds(s * RPS, RPS)],
                            out_hbm.at[r, c, pl.ds(s * RPS, RPS)])

            @pl.when(s == 0)
            def _(r=r):
                pltpu.sync_copy(acc.at[pl.ds(NS * RPS, ZREM)],
                                out_hbm.at[r, c, pl.ds(NS * RPS, ZREM)])

    return kern(tables, sd, zer_hbm)


def _tc_tables1(x, w1s, degt):
    def body(x_ref, w_ref, deg_ref, out_ref):
        xv = x_ref[...]
        for r in range(3):
            ns = _norm(deg_ref[:, r])
            t = jnp.dot(xv, w_ref[r], preferred_element_type=jnp.float32)
            t = t * ns[:, None]
            out_ref[r, 0] = t[:, : HID_F // 2]
            out_ref[r, 1] = t[:, HID_F // 2 :]

    return pl.pallas_call(
        body,
        grid=(N // BN,),
        in_specs=[
            pl.BlockSpec((BN, IN_F), lambda i: (i, 0)),
            pl.BlockSpec((3, IN_F, HID_F), lambda i: (0, 0, 0)),
            pl.BlockSpec((BN, 6), lambda i: (i, 0)),
        ],
        out_specs=pl.BlockSpec((3, 2, BN, HID_F // 2), lambda i: (0, 0, i, 0)),
        out_shape=jax.ShapeDtypeStruct((3, 2, N, HID_F // 2), jnp.float32),
    )(x, w1s, degt)


def _tc_mid(acc1, degt, w2s, b1s):
    def body(a_ref, deg_ref, w_ref, b_ref, out_ref):
        b1sum = b_ref[0] + b_ref[1] + b_ref[2]
        h0 = jnp.zeros((BN, HID_F // 2), jnp.float32)
        h1 = jnp.zeros((BN, HID_F // 2), jnp.float32)
        for r in range(3):
            nd = _norm(deg_ref[:, 3 + r])[:, None]
            h0 = h0 + a_ref[r, 0] * nd
            h1 = h1 + a_ref[r, 1] * nd
        h0 = jnp.maximum(h0 + b1sum[None, : HID_F // 2], 0.0)
        h1 = jnp.maximum(h1 + b1sum[None, HID_F // 2 :], 0.0)
        for r in range(3):
            ns = _norm(deg_ref[:, r])[:, None]
            t = (jnp.dot(h0, w_ref[r, : HID_F // 2], preferred_element_type=jnp.float32)
                 + jnp.dot(h1, w_ref[r, HID_F // 2 :], preferred_element_type=jnp.float32))
            out_ref[r] = t * ns

    return pl.pallas_call(
        body,
        grid=(N // BN,),
        in_specs=[
            pl.BlockSpec((3, 2, BN, HID_F // 2), lambda i: (0, 0, i, 0)),
            pl.BlockSpec((BN, 6), lambda i: (i, 0)),
            pl.BlockSpec((3, HID_F, OUT_F), lambda i: (0, 0, 0)),
            pl.BlockSpec((3, HID_F), lambda i: (0, 0)),
        ],
        out_specs=pl.BlockSpec((3, BN, OUT_F), lambda i: (0, i, 0)),
        out_shape=jax.ShapeDtypeStruct((3, N, OUT_F), jnp.float32),
    )(acc1, degt, w2s, b1s)


def _tc_final(acc2, degt, b2s):
    def body(a_ref, deg_ref, b_ref, out_ref):
        o = (b_ref[0] + b_ref[1] + b_ref[2])[None, :] * jnp.ones((BN, 1), jnp.float32)
        for r in range(3):
            nd = _norm(deg_ref[:, 3 + r])[:, None]
            o = o + (a_ref[r, 0] + a_ref[r, 1]) * nd
        out_ref[...] = o

    return pl.pallas_call(
        body,
        grid=(N // BN,),
        in_specs=[
            pl.BlockSpec((3, 2, BN, OUT_F), lambda i: (0, 0, i, 0)),
            pl.BlockSpec((BN, 6), lambda i: (i, 0)),
            pl.BlockSpec((3, OUT_F), lambda i: (0, 0)),
        ],
        out_specs=pl.BlockSpec((BN, OUT_F), lambda i: (i, 0)),
        out_shape=jax.ShapeDtypeStruct((N, OUT_F), jnp.float32),
    )(acc2, degt, b2s)


def kernel(x, edge_index0, edge_index1, edge_index2,
           W1_0, b1_0, W1_1, b1_1, W1_2, b1_2,
           W2_0, b2_0, W2_1, b2_1, W2_2, b2_2):
    src = jnp.concatenate([edge_index0[0], edge_index1[0], edge_index2[0]]).astype(jnp.int32)
    dst = jnp.concatenate([edge_index0[1], edge_index1[1], edge_index2[1]]).astype(jnp.int32)
    idx_flat = jnp.concatenate([src, dst])
    sd = jnp.stack([src.reshape(3 * TE, KE), dst.reshape(3 * TE, KE)], axis=1)

    w1s = jnp.stack([W1_0, W1_1, W1_2])
    b1s = jnp.stack([b1_0, b1_1, b1_2])
    w2s = jnp.stack([W2_0, W2_1, W2_2])
    b2s = jnp.stack([b2_0, b2_1, b2_2])

    zer_h1 = jnp.zeros((RPS, HID_F // NC), jnp.float32)
    zer_h2 = jnp.zeros((RPS, OUT_F), jnp.float32)
    assert RPS >= ZREM

    deg = _sc_bincount(idx_flat)[:, :, 0, :N]
    degt = deg.reshape(6, N).T
    t1 = _tc_tables1(x, w1s, degt)
    acc1 = _sc_edge(t1, sd, zer_h1, split_edges=False)
    t2 = _tc_mid(acc1, degt, w2s, b1s)
    acc2 = _sc_edge(t2, sd, zer_h2, split_edges=True)
    return _tc_final(acc2, degt, b2s)
